# Initial kernel scaffold; baseline (speedup 1.0000x reference)
import jax
import jax.numpy as jnp
from jax.experimental import pallas as pl


def kernel(x, edge_index, W_in, b_in, W_gcn, b_gcn, W_out, b_out):
    raise NotImplementedError("write your pallas kernel here")


# speedup vs baseline: 3.5656x; 3.5656x over previous
import jax
import jax.numpy as jnp
from jax.experimental import pallas as pl

N_BLK = 1000


def _final_body(h_ref, w_ref, b_ref, o_ref):
    o_ref[...] = jnp.maximum(h_ref[...], 0.0) @ w_ref[...] + b_ref[...]


def kernel(x, edge_index, W_in, b_in, W_gcn, b_gcn, W_out, b_out):
    N = x.shape[0]
    src = edge_index[0].astype(jnp.int32)
    dst = edge_index[1].astype(jnp.int32)
    h = jax.nn.relu(x @ W_in.T + b_in)
    deg = jnp.zeros((N,), x.dtype).at[dst].add(1.0) + 1.0
    dinv = jax.lax.rsqrt(deg)
    hw = h @ W_gcn.T
    g = hw * dinv[:, None]
    acc = jnp.zeros_like(g).at[dst].add(jnp.take(g, src, axis=0))
    pre = dinv[:, None] * (acc + g) + b_gcn

    nclass = W_out.shape[0]
    y = pl.pallas_call(
        _final_body,
        grid=(N // N_BLK,),
        in_specs=[
            pl.BlockSpec((N_BLK, 128), lambda i: (i, 0)),
            pl.BlockSpec((128, nclass), lambda i: (0, 0)),
            pl.BlockSpec((1, nclass), lambda i: (0, 0)),
        ],
        out_specs=pl.BlockSpec((N_BLK, nclass), lambda i: (i, 0)),
        out_shape=jax.ShapeDtypeStruct((N, nclass), x.dtype),
    )(pre, W_out.T, b_out.reshape(1, nclass))
    return y


# speedup vs baseline: 25.7570x; 7.2237x over previous
import functools

import jax
import jax.numpy as jnp
from jax import lax
from jax.experimental import pallas as pl
from jax.experimental.pallas import tpu as pltpu
from jax.experimental.pallas import tpu_sc as plsc

N = 10000
F = 128
E = 320000
NC = 2
NS = 16
NW = NC * NS
EPT = E // NW
CW = 80
NCHUNK = EPT // CW
NP = 10240
RPT = NP // NS
DEGW = 16

_mesh = plsc.VectorSubcoreMesh(core_axis_name="c", subcore_axis_name="s")


@functools.partial(
    pl.kernel,
    out_type=jax.ShapeDtypeStruct((NC, NP, DEGW), jnp.float32),
    mesh=_mesh,
    scratch_types=[
        pltpu.VMEM((NCHUNK, CW), jnp.int32),
        pltpu.VMEM((CW, DEGW), jnp.float32),
        pltpu.VMEM_SHARED((NP, DEGW), jnp.float32),
    ],
)
def _deg_kernel(dst_hbm, out_hbm, idx_v, ones_v, deg_sh):
    cid = lax.axis_index("c")
    sid = lax.axis_index("s")
    wid = sid * NC + cid
    pltpu.sync_copy(dst_hbm.at[wid], idx_v)
    ones16 = jnp.ones((DEGW,), jnp.float32)
    zeros16 = jnp.zeros((DEGW,), jnp.float32)

    def _fill_zero(i, _):
        ones_v[i, :] = zeros16
        return 0

    lax.fori_loop(0, CW, _fill_zero, 0)
    for j in range(RPT // CW):
        pltpu.sync_copy(ones_v, deg_sh.at[pl.ds(sid * RPT + j * CW, CW)])

    def _fill_ones(i, _):
        ones_v[i, :] = ones16
        return 0

    lax.fori_loop(0, CW, _fill_ones, 0)
    plsc.subcore_barrier()

    def _accum(c, _):
        pltpu.sync_copy(ones_v, deg_sh.at[idx_v.at[c]], add=True)
        return 0

    lax.fori_loop(0, NCHUNK, _accum, 0)
    plsc.subcore_barrier()
    pltpu.sync_copy(deg_sh.at[pl.ds(sid * RPT, RPT)],
                    out_hbm.at[cid, pl.ds(sid * RPT, RPT)])


@functools.partial(
    pl.kernel,
    out_type=jax.ShapeDtypeStruct((NC, NP, F), jnp.float32),
    mesh=_mesh,
    scratch_types=[
        pltpu.VMEM((NCHUNK, CW), jnp.int32),
        pltpu.VMEM((NCHUNK, CW), jnp.int32),
        pltpu.VMEM((CW, F), jnp.float32),
        pltpu.VMEM_SHARED((NP, F), jnp.float32),
        pltpu.SemaphoreType.DMA,
    ],
)
def _scatter_kernel(g_hbm, src_hbm, dst_hbm, out_hbm,
                    sidx_v, didx_v, rows_v, acc_sh, sem):
    cid = lax.axis_index("c")
    sid = lax.axis_index("s")
    wid = sid * NC + cid
    pltpu.sync_copy(src_hbm.at[wid], sidx_v)
    pltpu.sync_copy(dst_hbm.at[wid], didx_v)
    zeros16 = jnp.zeros((16,), jnp.float32)

    def _fill_zero(k, _):
        i = k // (F // 16)
        j = k % (F // 16)
        rows_v[i, pl.ds(j * 16, 16)] = zeros16
        return 0

    lax.fori_loop(0, CW * (F // 16), _fill_zero, 0)
    for j in range(RPT // CW):
        pltpu.sync_copy(rows_v, acc_sh.at[pl.ds(sid * RPT + j * CW, CW)])
    plsc.subcore_barrier()

    def _edge_chunk(c, _):
        pltpu.async_copy(g_hbm.at[sidx_v.at[c]], rows_v, sem).wait()
        pltpu.sync_copy(rows_v, acc_sh.at[didx_v.at[c]], add=True)
        return 0

    lax.fori_loop(0, NCHUNK, _edge_chunk, 0)
    plsc.subcore_barrier()
    pltpu.sync_copy(acc_sh.at[pl.ds(sid * RPT, RPT)],
                    out_hbm.at[cid, pl.ds(sid * RPT, RPT)])


def _mid_body(x_ref, wi_ref, bi_ref, wg_ref, d0_ref, d1_ref, g_ref, dinv_ref):
    h = jnp.maximum(
        jnp.dot(x_ref[...], wi_ref[...], preferred_element_type=jnp.float32)
        + bi_ref[...], 0.0)
    hw = jnp.dot(h, wg_ref[...], preferred_element_type=jnp.float32)
    deg = d0_ref[...] + d1_ref[...] + 1.0
    dinv = lax.rsqrt(deg)
    g_ref[...] = hw * dinv
    dinv_ref[...] = dinv


def _out_body(a_ref, g_ref, dinv_ref, bg_ref, wo_ref, bo_ref, o_ref):
    pre = (a_ref[0] + a_ref[1] + g_ref[...]) * dinv_ref[...] + bg_ref[...]
    o_ref[...] = jnp.dot(jnp.maximum(pre, 0.0), wo_ref[...],
                         preferred_element_type=jnp.float32) + bo_ref[...]


N_BLK = 1000


def kernel(x, edge_index, W_in, b_in, W_gcn, b_gcn, W_out, b_out):
    nclass = W_out.shape[0]
    src = edge_index[0].astype(jnp.int32).reshape(NW, NCHUNK, CW)
    dst = edge_index[1].astype(jnp.int32).reshape(NW, NCHUNK, CW)

    degp = _deg_kernel(dst)
    d0 = degp[0, :N, 0:1]
    d1 = degp[1, :N, 0:1]

    g, dinv = pl.pallas_call(
        _mid_body,
        grid=(N // N_BLK,),
        in_specs=[
            pl.BlockSpec((N_BLK, F), lambda i: (i, 0)),
            pl.BlockSpec((F, F), lambda i: (0, 0)),
            pl.BlockSpec((1, F), lambda i: (0, 0)),
            pl.BlockSpec((F, F), lambda i: (0, 0)),
            pl.BlockSpec((N_BLK, 1), lambda i: (i, 0)),
            pl.BlockSpec((N_BLK, 1), lambda i: (i, 0)),
        ],
        out_specs=[
            pl.BlockSpec((N_BLK, F), lambda i: (i, 0)),
            pl.BlockSpec((N_BLK, 1), lambda i: (i, 0)),
        ],
        out_shape=[
            jax.ShapeDtypeStruct((N, F), jnp.float32),
            jax.ShapeDtypeStruct((N, 1), jnp.float32),
        ],
    )(x, W_in.T, b_in.reshape(1, F), W_gcn.T, d0, d1)

    acc = _scatter_kernel(g, src, dst)[:, :N, :]

    y = pl.pallas_call(
        _out_body,
        grid=(N // N_BLK,),
        in_specs=[
            pl.BlockSpec((NC, N_BLK, F), lambda i: (0, i, 0)),
            pl.BlockSpec((N_BLK, F), lambda i: (i, 0)),
            pl.BlockSpec((N_BLK, 1), lambda i: (i, 0)),
            pl.BlockSpec((1, F), lambda i: (0, 0)),
            pl.BlockSpec((F, nclass), lambda i: (0, 0)),
            pl.BlockSpec((1, nclass), lambda i: (0, 0)),
        ],
        out_specs=pl.BlockSpec((N_BLK, nclass), lambda i: (i, 0)),
        out_shape=jax.ShapeDtypeStruct((N, nclass), jnp.float32),
    )(acc, g, dinv, b_gcn.reshape(1, F), W_out.T, b_out.reshape(1, nclass))
    return y
